# Initial kernel scaffold; baseline (speedup 1.0000x reference)
import jax
import jax.numpy as jnp
from jax.experimental import pallas as pl


def kernel(x, edge_index, edge_weight, W1, b1, W2, b2, Wc, bc):
    raise NotImplementedError("write your pallas kernel here")


# speedup vs baseline: 4.6244x; 4.6244x over previous
import functools

import jax
import jax.numpy as jnp
from jax import lax
from jax.experimental import pallas as pl
from jax.experimental.pallas import tpu as pltpu
from jax.experimental.pallas import tpu_sc as plsc

N = 10000
E = 320000
DIN = 128
H = 256
C = 64

NP = 10240
HF = H // 2
K = 128
NSC = 2
NTILE = 16
EPT = 20480
CH = EPT // K
BCH = 32
NBLK = CH // BCH
E_PAD = NTILE * EPT
ECH = E_PAD // K
SLAB = NP // NTILE
RB = 512
NRB = NP // RB


_mesh = plsc.VectorSubcoreMesh(core_axis_name="c", subcore_axis_name="s")


@functools.partial(
    pl.kernel,
    mesh=_mesh,
    out_type=jax.ShapeDtypeStruct((2 * NP, HF), jnp.float32),
    scratch_types=[
        pltpu.VMEM((BCH, K), jnp.int32),
        pltpu.VMEM((BCH, K), jnp.int32),
        pltpu.VMEM((BCH, K), jnp.float32),
        pltpu.VMEM((K, HF), jnp.float32),
        pltpu.VMEM_SHARED((NP, HF), jnp.float32),
    ],
)
def _propagate(h_hbm, src_hbm, dst_hbm, w_hbm, out_hbm,
               src_v, dst_v, w_v, rows_v, acc_sh):
    c = lax.axis_index("c")
    s = lax.axis_index("s")

    @pl.loop(0, K)
    def _(e):
        for f in range(0, HF, 16):
            rows_v[e, pl.ds(f, 16)] = jnp.zeros((16,), jnp.float32)

    for i in range(SLAB // K):
        pltpu.sync_copy(rows_v, acc_sh.at[pl.ds(s * SLAB + i * K, K)])

    off = c * NP

    plsc.subcore_barrier()

    @pl.loop(0, NBLK)
    def _(t):
        base = s * CH + t * BCH
        pltpu.sync_copy(src_hbm.at[pl.ds(base, BCH)], src_v)
        pltpu.sync_copy(dst_hbm.at[pl.ds(base, BCH)], dst_v)
        pltpu.sync_copy(w_hbm.at[pl.ds(base, BCH)], w_v)

        @pl.loop(0, BCH)
        def _(j):
            @pl.loop(0, K, step=16)
            def _(e):
                src_v[j, pl.ds(e, 16)] = src_v[j, pl.ds(e, 16)] + off

        @pl.loop(0, BCH)
        def _(j):
            pltpu.sync_copy(h_hbm.at[src_v.at[j]], rows_v)

            @pl.loop(0, K, step=16)
            def _(e0):
                wv = w_v[j, pl.ds(e0, 16)]
                for i in range(16):
                    we = wv[i]
                    for f in range(0, HF, 16):
                        rows_v[e0 + i, pl.ds(f, 16)] = (
                            rows_v[e0 + i, pl.ds(f, 16)] * we)

            pltpu.sync_copy(rows_v, acc_sh.at[dst_v.at[j]], add=True)

    plsc.subcore_barrier()

    pltpu.sync_copy(acc_sh.at[pl.ds(s * SLAB, SLAB)],
                    out_hbm.at[pl.ds(off + s * SLAB, SLAB)])


def _mm_body(x_ref, w_ref, o_ref):
    o_ref[...] = jnp.dot(x_ref[...], w_ref[...],
                         preferred_element_type=jnp.float32)


def _in_matmul(x_pd, W1):
    return pl.pallas_call(
        _mm_body,
        grid=(NRB, 2),
        in_specs=[
            pl.BlockSpec((RB, DIN), lambda i, j: (i, 0)),
            pl.BlockSpec((DIN, HF), lambda i, j: (0, j)),
        ],
        out_specs=pl.BlockSpec((RB, HF), lambda i, j: (i + j * NRB, 0)),
        out_shape=jax.ShapeDtypeStruct((2 * NP, HF), jnp.float32),
    )(x_pd, W1)


def _mid_body(alo_ref, ahi_ref, b_ref, wt_ref, wb_ref, o_ref):
    b = b_ref[...]
    zlo = jnp.maximum(alo_ref[...] + b[:, :HF], 0.0)
    zhi = jnp.maximum(ahi_ref[...] + b[:, HF:], 0.0)
    o_ref[...] = (jnp.dot(zlo, wt_ref[...], preferred_element_type=jnp.float32)
                  + jnp.dot(zhi, wb_ref[...], preferred_element_type=jnp.float32))


def _mid_matmul(agg, b1, W2):
    return pl.pallas_call(
        _mid_body,
        grid=(NRB, 2),
        in_specs=[
            pl.BlockSpec((RB, HF), lambda i, j: (i, 0)),
            pl.BlockSpec((RB, HF), lambda i, j: (i + NRB, 0)),
            pl.BlockSpec((1, H), lambda i, j: (0, 0)),
            pl.BlockSpec((HF, HF), lambda i, j: (0, j)),
            pl.BlockSpec((HF, HF), lambda i, j: (1, j)),
        ],
        out_specs=pl.BlockSpec((RB, HF), lambda i, j: (i + j * NRB, 0)),
        out_shape=jax.ShapeDtypeStruct((2 * NP, HF), jnp.float32),
    )(agg, agg, b1.reshape(1, H), W2, W2)


def _head_body(alo_ref, ahi_ref, b_ref, wt_ref, wb_ref, bc_ref, lp_ref, pr_ref):
    b = b_ref[...]
    zlo = jnp.maximum(alo_ref[...] + b[:, :HF], 0.0)
    zhi = jnp.maximum(ahi_ref[...] + b[:, HF:], 0.0)
    logits = (jnp.dot(zlo, wt_ref[...], preferred_element_type=jnp.float32)
              + jnp.dot(zhi, wb_ref[...], preferred_element_type=jnp.float32)
              + bc_ref[...])
    m = jnp.max(logits, axis=1, keepdims=True)
    ex = jnp.exp(logits - m)
    se = jnp.sum(ex, axis=1, keepdims=True)
    lp_ref[...] = logits - m - jnp.log(se)
    pr_ref[...] = ex / se


def _head(agg, b2, WcT, bc):
    return pl.pallas_call(
        _head_body,
        grid=(NRB,),
        in_specs=[
            pl.BlockSpec((RB, HF), lambda i: (i, 0)),
            pl.BlockSpec((RB, HF), lambda i: (i + NRB, 0)),
            pl.BlockSpec((1, H), lambda i: (0, 0)),
            pl.BlockSpec((HF, C), lambda i: (0, 0)),
            pl.BlockSpec((HF, C), lambda i: (1, 0)),
            pl.BlockSpec((1, C), lambda i: (0, 0)),
        ],
        out_specs=[
            pl.BlockSpec((RB, C), lambda i: (i, 0)),
            pl.BlockSpec((RB, C), lambda i: (i, 0)),
        ],
        out_shape=[
            jax.ShapeDtypeStruct((NP, C), jnp.float32),
            jax.ShapeDtypeStruct((NP, C), jnp.float32),
        ],
    )(agg, agg, b2.reshape(1, H), WcT, WcT, bc.reshape(1, C))


def kernel(x, edge_index, edge_weight, W1, b1, W2, b2, Wc, bc):
    src = edge_index[0]
    dst = edge_index[1]
    pad = E_PAD - E
    fill = jnp.arange(pad, dtype=jnp.int32) % N
    src_p = jnp.concatenate([src, fill]).reshape(ECH, K)
    dst_p = jnp.concatenate([dst, fill]).reshape(ECH, K)
    w_p = jnp.concatenate(
        [edge_weight, jnp.zeros((pad,), jnp.float32)]).reshape(ECH, K)
    x_pd = jnp.pad(x, ((0, NP - N), (0, 0)))

    h1 = _in_matmul(x_pd, W1)
    agg1 = _propagate(h1, src_p, dst_p, w_p)
    h2 = _mid_matmul(agg1, b1, W2)
    agg2 = _propagate(h2, src_p, dst_p, w_p)
    lp, pr = _head(agg2, b2, Wc.T, bc)
    return lp[:N], pr[:N]


# speedup vs baseline: 6.8582x; 1.4830x over previous
import functools

import jax
import jax.numpy as jnp
from jax import lax
from jax.experimental import pallas as pl
from jax.experimental.pallas import tpu as pltpu
from jax.experimental.pallas import tpu_sc as plsc

N = 10000
E = 320000
DIN = 128
H = 256
C = 64

NP = 10240
HF = H // 2
K = 128
NSC = 2
NTILE = 16
EPT = 20480
CH = EPT // K
BCH = 16
NBLK = CH // BCH
E_PAD = NTILE * EPT
ECH = E_PAD // K
SLAB = NP // NTILE
RB = 512
NRB = NP // RB


_mesh = plsc.VectorSubcoreMesh(core_axis_name="c", subcore_axis_name="s")


@functools.partial(
    pl.kernel,
    mesh=_mesh,
    out_type=jax.ShapeDtypeStruct((2 * NP, HF), jnp.float32),
    scratch_types=[
        pltpu.VMEM((BCH, K), jnp.int32),
        pltpu.VMEM((BCH, K), jnp.int32),
        pltpu.VMEM((BCH, K), jnp.float32),
        pltpu.VMEM((K, HF), jnp.float32),
        pltpu.VMEM((K, HF), jnp.float32),
        pltpu.VMEM_SHARED((NP, HF), jnp.float32),
        pltpu.SemaphoreType.DMA,
        pltpu.SemaphoreType.DMA,
        pltpu.SemaphoreType.DMA,
        pltpu.SemaphoreType.DMA,
    ],
)
def _propagate(h_hbm, src_hbm, dst_hbm, w_hbm, out_hbm,
               src_v, dst_v, w_v, rows0, rows1, acc_sh,
               sg0, sg1, ss0, ss1):
    c = lax.axis_index("c")
    s = lax.axis_index("s")

    def _scale(rows, j):
        @pl.loop(0, K, step=16)
        def _(e0):
            wv = w_v[j, pl.ds(e0, 16)]
            for i in range(16):
                we = wv[i]
                for f in range(0, HF, 16):
                    rows[e0 + i, pl.ds(f, 16)] = (
                        rows[e0 + i, pl.ds(f, 16)] * we)

    @pl.loop(0, K)
    def _(e):
        for f in range(0, HF, 16):
            rows0[e, pl.ds(f, 16)] = jnp.zeros((16,), jnp.float32)

    for i in range(SLAB // K):
        pltpu.sync_copy(rows0, acc_sh.at[pl.ds(s * SLAB + i * K, K)])

    off = c * NP

    plsc.subcore_barrier()

    @pl.loop(0, NBLK)
    def _(t):
        base = s * CH + t * BCH
        pltpu.sync_copy(src_hbm.at[pl.ds(base, BCH)], src_v)
        pltpu.sync_copy(dst_hbm.at[pl.ds(base, BCH)], dst_v)
        pltpu.sync_copy(w_hbm.at[pl.ds(base, BCH)], w_v)

        @pl.loop(0, BCH)
        def _(j):
            @pl.loop(0, K, step=16)
            def _(e):
                src_v[j, pl.ds(e, 16)] = src_v[j, pl.ds(e, 16)] + off

        pltpu.async_copy(h_hbm.at[src_v.at[0]], rows0, sg0)

        @pl.loop(0, BCH, step=2)
        def _(j):
            @pl.when(j > 0)
            def _():
                pltpu.make_async_copy(rows1, acc_sh.at[dst_v.at[j]], ss1).wait()
            pltpu.async_copy(h_hbm.at[src_v.at[j + 1]], rows1, sg1)
            pltpu.make_async_copy(h_hbm.at[src_v.at[j]], rows0, sg0).wait()
            _scale(rows0, j)
            pltpu.async_copy(rows0, acc_sh.at[dst_v.at[j]], ss0, add=True)

            @pl.when(j + 2 < BCH)
            def _():
                pltpu.make_async_copy(rows0, acc_sh.at[dst_v.at[j]], ss0).wait()
                pltpu.async_copy(h_hbm.at[src_v.at[j + 2]], rows0, sg0)
            pltpu.make_async_copy(h_hbm.at[src_v.at[j + 1]], rows1, sg1).wait()
            _scale(rows1, j + 1)
            pltpu.async_copy(rows1, acc_sh.at[dst_v.at[j + 1]], ss1, add=True)

        pltpu.make_async_copy(rows0, acc_sh.at[dst_v.at[BCH - 2]], ss0).wait()
        pltpu.make_async_copy(rows1, acc_sh.at[dst_v.at[BCH - 1]], ss1).wait()

    plsc.subcore_barrier()

    pltpu.sync_copy(acc_sh.at[pl.ds(s * SLAB, SLAB)],
                    out_hbm.at[pl.ds(off + s * SLAB, SLAB)])


def _mm_body(x_ref, w_ref, o_ref):
    o_ref[...] = jnp.dot(x_ref[...], w_ref[...],
                         preferred_element_type=jnp.float32)


def _in_matmul(x_pd, W1):
    return pl.pallas_call(
        _mm_body,
        grid=(NRB, 2),
        in_specs=[
            pl.BlockSpec((RB, DIN), lambda i, j: (i, 0)),
            pl.BlockSpec((DIN, HF), lambda i, j: (0, j)),
        ],
        out_specs=pl.BlockSpec((RB, HF), lambda i, j: (i + j * NRB, 0)),
        out_shape=jax.ShapeDtypeStruct((2 * NP, HF), jnp.float32),
    )(x_pd, W1)


def _mid_body(alo_ref, ahi_ref, b_ref, wt_ref, wb_ref, o_ref):
    b = b_ref[...]
    zlo = jnp.maximum(alo_ref[...] + b[:, :HF], 0.0)
    zhi = jnp.maximum(ahi_ref[...] + b[:, HF:], 0.0)
    o_ref[...] = (jnp.dot(zlo, wt_ref[...], preferred_element_type=jnp.float32)
                  + jnp.dot(zhi, wb_ref[...], preferred_element_type=jnp.float32))


def _mid_matmul(agg, b1, W2):
    return pl.pallas_call(
        _mid_body,
        grid=(NRB, 2),
        in_specs=[
            pl.BlockSpec((RB, HF), lambda i, j: (i, 0)),
            pl.BlockSpec((RB, HF), lambda i, j: (i + NRB, 0)),
            pl.BlockSpec((1, H), lambda i, j: (0, 0)),
            pl.BlockSpec((HF, HF), lambda i, j: (0, j)),
            pl.BlockSpec((HF, HF), lambda i, j: (1, j)),
        ],
        out_specs=pl.BlockSpec((RB, HF), lambda i, j: (i + j * NRB, 0)),
        out_shape=jax.ShapeDtypeStruct((2 * NP, HF), jnp.float32),
    )(agg, agg, b1.reshape(1, H), W2, W2)


def _head_body(alo_ref, ahi_ref, b_ref, wt_ref, wb_ref, bc_ref, lp_ref, pr_ref):
    b = b_ref[...]
    zlo = jnp.maximum(alo_ref[...] + b[:, :HF], 0.0)
    zhi = jnp.maximum(ahi_ref[...] + b[:, HF:], 0.0)
    logits = (jnp.dot(zlo, wt_ref[...], preferred_element_type=jnp.float32)
              + jnp.dot(zhi, wb_ref[...], preferred_element_type=jnp.float32)
              + bc_ref[...])
    m = jnp.max(logits, axis=1, keepdims=True)
    ex = jnp.exp(logits - m)
    se = jnp.sum(ex, axis=1, keepdims=True)
    lp_ref[...] = logits - m - jnp.log(se)
    pr_ref[...] = ex / se


def _head(agg, b2, WcT, bc):
    return pl.pallas_call(
        _head_body,
        grid=(NRB,),
        in_specs=[
            pl.BlockSpec((RB, HF), lambda i: (i, 0)),
            pl.BlockSpec((RB, HF), lambda i: (i + NRB, 0)),
            pl.BlockSpec((1, H), lambda i: (0, 0)),
            pl.BlockSpec((HF, C), lambda i: (0, 0)),
            pl.BlockSpec((HF, C), lambda i: (1, 0)),
            pl.BlockSpec((1, C), lambda i: (0, 0)),
        ],
        out_specs=[
            pl.BlockSpec((RB, C), lambda i: (i, 0)),
            pl.BlockSpec((RB, C), lambda i: (i, 0)),
        ],
        out_shape=[
            jax.ShapeDtypeStruct((NP, C), jnp.float32),
            jax.ShapeDtypeStruct((NP, C), jnp.float32),
        ],
    )(agg, agg, b2.reshape(1, H), WcT, WcT, bc.reshape(1, C))


def kernel(x, edge_index, edge_weight, W1, b1, W2, b2, Wc, bc):
    src = edge_index[0]
    dst = edge_index[1]
    pad = E_PAD - E
    fill = jnp.arange(pad, dtype=jnp.int32) % N
    src_p = jnp.concatenate([src, fill]).reshape(ECH, K)
    dst_p = jnp.concatenate([dst, fill]).reshape(ECH, K)
    w_p = jnp.concatenate(
        [edge_weight, jnp.zeros((pad,), jnp.float32)]).reshape(ECH, K)
    x_pd = jnp.pad(x, ((0, NP - N), (0, 0)))

    h1 = _in_matmul(x_pd, W1)
    agg1 = _propagate(h1, src_p, dst_p, w_p)
    h2 = _mid_matmul(agg1, b1, W2)
    agg2 = _propagate(h2, src_p, dst_p, w_p)
    lp, pr = _head(agg2, b2, Wc.T, bc)
    return lp[:N], pr[:N]


# speedup vs baseline: 7.2446x; 1.0563x over previous
import functools

import jax
import jax.numpy as jnp
from jax import lax
from jax.experimental import pallas as pl
from jax.experimental.pallas import tpu as pltpu
from jax.experimental.pallas import tpu_sc as plsc

N = 10000
E = 320000
DIN = 128
H = 256
C = 64

NP = 10240
HF = H // 2
K = 128
NSC = 2
NTILE = 16
EPT = 20480
CH = EPT // K
BCH = 32
NBLK = CH // BCH
E_PAD = NTILE * EPT
ECH = E_PAD // K
SLAB = NP // NTILE
RB = 512
NRB = NP // RB


_mesh = plsc.VectorSubcoreMesh(core_axis_name="c", subcore_axis_name="s")


@functools.partial(
    pl.kernel,
    mesh=_mesh,
    out_type=jax.ShapeDtypeStruct((2 * NP, HF), jnp.float32),
    scratch_types=[
        pltpu.VMEM((BCH, K), jnp.int32),
        pltpu.VMEM((BCH, K), jnp.int32),
        pltpu.VMEM((BCH, K), jnp.float32),
        pltpu.VMEM((K, HF), jnp.float32),
        pltpu.VMEM((K, HF), jnp.float32),
        pltpu.VMEM_SHARED((NP, HF), jnp.float32),
        pltpu.SemaphoreType.DMA,
        pltpu.SemaphoreType.DMA,
        pltpu.SemaphoreType.DMA,
        pltpu.SemaphoreType.DMA,
    ],
)
def _propagate(h_hbm, src_hbm, dst_hbm, w_hbm, out_hbm,
               src_v, dst_v, w_v, rows0, rows1, acc_sh,
               sg0, sg1, ss0, ss1):
    c = lax.axis_index("c")
    s = lax.axis_index("s")

    def _scale(rows, j):
        @plsc.parallel_loop(0, K, step=16, unroll=2)
        def _(e0):
            wv = w_v[j, pl.ds(e0, 16)]
            for i in range(16):
                we = wv[i]
                for f in range(0, HF, 16):
                    rows[e0 + i, pl.ds(f, 16)] = (
                        rows[e0 + i, pl.ds(f, 16)] * we)

    @pl.loop(0, K)
    def _(e):
        for f in range(0, HF, 16):
            rows0[e, pl.ds(f, 16)] = jnp.zeros((16,), jnp.float32)

    for i in range(SLAB // K):
        pltpu.sync_copy(rows0, acc_sh.at[pl.ds(s * SLAB + i * K, K)])

    off = c * NP

    plsc.subcore_barrier()

    @pl.loop(0, NBLK)
    def _(t):
        base = s * CH + t * BCH
        pltpu.sync_copy(src_hbm.at[pl.ds(base, BCH)], src_v)
        pltpu.sync_copy(dst_hbm.at[pl.ds(base, BCH)], dst_v)
        pltpu.sync_copy(w_hbm.at[pl.ds(base, BCH)], w_v)

        @pl.loop(0, BCH)
        def _(j):
            @plsc.parallel_loop(0, K, step=16, unroll=2)
            def _(e):
                src_v[j, pl.ds(e, 16)] = src_v[j, pl.ds(e, 16)] + off

        pltpu.async_copy(h_hbm.at[src_v.at[0]], rows0, sg0)

        @pl.loop(0, BCH, step=2)
        def _(j):
            @pl.when(j > 0)
            def _():
                pltpu.make_async_copy(rows1, acc_sh.at[dst_v.at[j]], ss1).wait()
            pltpu.async_copy(h_hbm.at[src_v.at[j + 1]], rows1, sg1)
            pltpu.make_async_copy(h_hbm.at[src_v.at[j]], rows0, sg0).wait()
            _scale(rows0, j)
            pltpu.async_copy(rows0, acc_sh.at[dst_v.at[j]], ss0, add=True)

            @pl.when(j + 2 < BCH)
            def _():
                pltpu.make_async_copy(rows0, acc_sh.at[dst_v.at[j]], ss0).wait()
                pltpu.async_copy(h_hbm.at[src_v.at[j + 2]], rows0, sg0)
            pltpu.make_async_copy(h_hbm.at[src_v.at[j + 1]], rows1, sg1).wait()
            _scale(rows1, j + 1)
            pltpu.async_copy(rows1, acc_sh.at[dst_v.at[j + 1]], ss1, add=True)

        pltpu.make_async_copy(rows0, acc_sh.at[dst_v.at[BCH - 2]], ss0).wait()
        pltpu.make_async_copy(rows1, acc_sh.at[dst_v.at[BCH - 1]], ss1).wait()

    plsc.subcore_barrier()

    pltpu.sync_copy(acc_sh.at[pl.ds(s * SLAB, SLAB)],
                    out_hbm.at[pl.ds(off + s * SLAB, SLAB)])


def _mm_body(x_ref, w_ref, o_ref):
    o_ref[...] = jnp.dot(x_ref[...], w_ref[...],
                         preferred_element_type=jnp.float32)


def _in_matmul(x_pd, W1):
    return pl.pallas_call(
        _mm_body,
        grid=(NRB, 2),
        in_specs=[
            pl.BlockSpec((RB, DIN), lambda i, j: (i, 0)),
            pl.BlockSpec((DIN, HF), lambda i, j: (0, j)),
        ],
        out_specs=pl.BlockSpec((RB, HF), lambda i, j: (i + j * NRB, 0)),
        out_shape=jax.ShapeDtypeStruct((2 * NP, HF), jnp.float32),
    )(x_pd, W1)


def _mid_body(alo_ref, ahi_ref, b_ref, wt_ref, wb_ref, o_ref):
    b = b_ref[...]
    zlo = jnp.maximum(alo_ref[...] + b[:, :HF], 0.0)
    zhi = jnp.maximum(ahi_ref[...] + b[:, HF:], 0.0)
    o_ref[...] = (jnp.dot(zlo, wt_ref[...], preferred_element_type=jnp.float32)
                  + jnp.dot(zhi, wb_ref[...], preferred_element_type=jnp.float32))


def _mid_matmul(agg, b1, W2):
    return pl.pallas_call(
        _mid_body,
        grid=(NRB, 2),
        in_specs=[
            pl.BlockSpec((RB, HF), lambda i, j: (i, 0)),
            pl.BlockSpec((RB, HF), lambda i, j: (i + NRB, 0)),
            pl.BlockSpec((1, H), lambda i, j: (0, 0)),
            pl.BlockSpec((HF, HF), lambda i, j: (0, j)),
            pl.BlockSpec((HF, HF), lambda i, j: (1, j)),
        ],
        out_specs=pl.BlockSpec((RB, HF), lambda i, j: (i + j * NRB, 0)),
        out_shape=jax.ShapeDtypeStruct((2 * NP, HF), jnp.float32),
    )(agg, agg, b1.reshape(1, H), W2, W2)


def _head_body(alo_ref, ahi_ref, b_ref, wt_ref, wb_ref, bc_ref, lp_ref, pr_ref):
    b = b_ref[...]
    zlo = jnp.maximum(alo_ref[...] + b[:, :HF], 0.0)
    zhi = jnp.maximum(ahi_ref[...] + b[:, HF:], 0.0)
    logits = (jnp.dot(zlo, wt_ref[...], preferred_element_type=jnp.float32)
              + jnp.dot(zhi, wb_ref[...], preferred_element_type=jnp.float32)
              + bc_ref[...])
    m = jnp.max(logits, axis=1, keepdims=True)
    ex = jnp.exp(logits - m)
    se = jnp.sum(ex, axis=1, keepdims=True)
    lp_ref[...] = logits - m - jnp.log(se)
    pr_ref[...] = ex / se


def _head(agg, b2, WcT, bc):
    return pl.pallas_call(
        _head_body,
        grid=(NRB,),
        in_specs=[
            pl.BlockSpec((RB, HF), lambda i: (i, 0)),
            pl.BlockSpec((RB, HF), lambda i: (i + NRB, 0)),
            pl.BlockSpec((1, H), lambda i: (0, 0)),
            pl.BlockSpec((HF, C), lambda i: (0, 0)),
            pl.BlockSpec((HF, C), lambda i: (1, 0)),
            pl.BlockSpec((1, C), lambda i: (0, 0)),
        ],
        out_specs=[
            pl.BlockSpec((RB, C), lambda i: (i, 0)),
            pl.BlockSpec((RB, C), lambda i: (i, 0)),
        ],
        out_shape=[
            jax.ShapeDtypeStruct((NP, C), jnp.float32),
            jax.ShapeDtypeStruct((NP, C), jnp.float32),
        ],
    )(agg, agg, b2.reshape(1, H), WcT, WcT, bc.reshape(1, C))


def kernel(x, edge_index, edge_weight, W1, b1, W2, b2, Wc, bc):
    src = edge_index[0]
    dst = edge_index[1]
    pad = E_PAD - E
    fill = jnp.arange(pad, dtype=jnp.int32) % N
    src_p = jnp.concatenate([src, fill]).reshape(ECH, K)
    dst_p = jnp.concatenate([dst, fill]).reshape(ECH, K)
    w_p = jnp.concatenate(
        [edge_weight, jnp.zeros((pad,), jnp.float32)]).reshape(ECH, K)
    x_pd = jnp.pad(x, ((0, NP - N), (0, 0)))

    h1 = _in_matmul(x_pd, W1)
    agg1 = _propagate(h1, src_p, dst_p, w_p)
    h2 = _mid_matmul(agg1, b1, W2)
    agg2 = _propagate(h2, src_p, dst_p, w_p)
    lp, pr = _head(agg2, b2, Wc.T, bc)
    return lp[:N], pr[:N]


# speedup vs baseline: 7.4528x; 1.0287x over previous
import functools

import jax
import jax.numpy as jnp
from jax import lax
from jax.experimental import pallas as pl
from jax.experimental.pallas import tpu as pltpu
from jax.experimental.pallas import tpu_sc as plsc

N = 10000
E = 320000
DIN = 128
H = 256
C = 64

NP = 10240
HF = H // 2
K = 96
NSC = 2
NTILE = 16
EPT = 20736
CH = EPT // K
BCH = 24
NBLK = CH // BCH
E_PAD = NTILE * EPT
ECH = E_PAD // K
SLAB = NP // NTILE
RB = 512
NRB = NP // RB


_mesh = plsc.VectorSubcoreMesh(core_axis_name="c", subcore_axis_name="s")


@functools.partial(
    pl.kernel,
    mesh=_mesh,
    out_type=jax.ShapeDtypeStruct((2 * NP, HF), jnp.float32),
    scratch_types=[
        pltpu.VMEM((BCH, K), jnp.int32),
        pltpu.VMEM((BCH, K), jnp.int32),
        pltpu.VMEM((BCH, K), jnp.float32),
        pltpu.VMEM((K, HF), jnp.float32),
        pltpu.VMEM((K, HF), jnp.float32),
        pltpu.VMEM((K, HF), jnp.float32),
        pltpu.VMEM_SHARED((NP, HF), jnp.float32),
        pltpu.SemaphoreType.DMA,
        pltpu.SemaphoreType.DMA,
        pltpu.SemaphoreType.DMA,
        pltpu.SemaphoreType.DMA,
        pltpu.SemaphoreType.DMA,
        pltpu.SemaphoreType.DMA,
    ],
)
def _propagate(h_hbm, src_hbm, dst_hbm, w_hbm, out_hbm,
               src_v, dst_v, w_v, rows0, rows1, rows2, acc_sh,
               sg0, sg1, sg2, ss0, ss1, ss2):
    c = lax.axis_index("c")
    s = lax.axis_index("s")

    def _scale(rows, j):
        @plsc.parallel_loop(0, K, step=16, unroll=3)
        def _(e0):
            wv = w_v[j, pl.ds(e0, 16)]
            for i in range(16):
                we = wv[i]
                for f in range(0, HF, 16):
                    rows[e0 + i, pl.ds(f, 16)] = (
                        rows[e0 + i, pl.ds(f, 16)] * we)

    @pl.loop(0, K)
    def _(e):
        for f in range(0, HF, 16):
            rows0[e, pl.ds(f, 16)] = jnp.zeros((16,), jnp.float32)

    for i in range(SLAB // K):
        pltpu.sync_copy(rows0, acc_sh.at[pl.ds(s * SLAB + i * K, K)])
    _rem = SLAB - (SLAB // K) * K
    if _rem:
        pltpu.sync_copy(rows0.at[pl.ds(0, _rem)],
                        acc_sh.at[pl.ds(s * SLAB + (SLAB // K) * K, _rem)])

    off = c * NP

    plsc.subcore_barrier()

    @pl.loop(0, NBLK)
    def _(t):
        base = s * CH + t * BCH
        pltpu.sync_copy(src_hbm.at[c, pl.ds(base, BCH)], src_v)
        pltpu.sync_copy(dst_hbm.at[pl.ds(base, BCH)], dst_v)
        pltpu.sync_copy(w_hbm.at[pl.ds(base, BCH)], w_v)

        bufs = ((rows0, sg0, ss0), (rows1, sg1, ss1), (rows2, sg2, ss2))
        pltpu.async_copy(h_hbm.at[src_v.at[0]], rows0, sg0)
        pltpu.async_copy(h_hbm.at[src_v.at[1]], rows1, sg1)

        @pl.loop(0, BCH, step=3)
        def _(j):
            for b in range(3):
                m = j + b
                rb, sgb, ssb = bufs[b]
                rn, sgn, ssn = bufs[(b + 2) % 3]
                pltpu.make_async_copy(h_hbm.at[src_v.at[m]], rb, sgb).wait()
                _scale(rb, m)
                pltpu.async_copy(rb, acc_sh.at[dst_v.at[m]], ssb, add=True)

                @pl.when(m + 2 < BCH)
                def _(m=m, rn=rn, sgn=sgn, ssn=ssn):
                    @pl.when(m > 0)
                    def _():
                        pltpu.make_async_copy(
                            rn, acc_sh.at[dst_v.at[m]], ssn).wait()
                    pltpu.async_copy(h_hbm.at[src_v.at[m + 2]], rn, sgn)

        pltpu.make_async_copy(rows0, acc_sh.at[dst_v.at[BCH - 3]], ss0).wait()
        pltpu.make_async_copy(rows1, acc_sh.at[dst_v.at[BCH - 2]], ss1).wait()
        pltpu.make_async_copy(rows2, acc_sh.at[dst_v.at[BCH - 1]], ss2).wait()

    plsc.subcore_barrier()

    pltpu.sync_copy(acc_sh.at[pl.ds(s * SLAB, SLAB)],
                    out_hbm.at[pl.ds(off + s * SLAB, SLAB)])


def _mm_body(x_ref, w_ref, o_ref):
    o_ref[...] = jnp.dot(x_ref[...], w_ref[...],
                         preferred_element_type=jnp.float32)


def _in_matmul(x_pd, W1):
    return pl.pallas_call(
        _mm_body,
        grid=(NRB, 2),
        in_specs=[
            pl.BlockSpec((RB, DIN), lambda i, j: (i, 0)),
            pl.BlockSpec((DIN, HF), lambda i, j: (0, j)),
        ],
        out_specs=pl.BlockSpec((RB, HF), lambda i, j: (i + j * NRB, 0)),
        out_shape=jax.ShapeDtypeStruct((2 * NP, HF), jnp.float32),
    )(x_pd, W1)


def _mid_body(alo_ref, ahi_ref, b_ref, wt_ref, wb_ref, o_ref):
    b = b_ref[...]
    zlo = jnp.maximum(alo_ref[...] + b[:, :HF], 0.0)
    zhi = jnp.maximum(ahi_ref[...] + b[:, HF:], 0.0)
    o_ref[...] = (jnp.dot(zlo, wt_ref[...], preferred_element_type=jnp.float32)
                  + jnp.dot(zhi, wb_ref[...], preferred_element_type=jnp.float32))


def _mid_matmul(agg, b1, W2):
    return pl.pallas_call(
        _mid_body,
        grid=(NRB, 2),
        in_specs=[
            pl.BlockSpec((RB, HF), lambda i, j: (i, 0)),
            pl.BlockSpec((RB, HF), lambda i, j: (i + NRB, 0)),
            pl.BlockSpec((1, H), lambda i, j: (0, 0)),
            pl.BlockSpec((HF, HF), lambda i, j: (0, j)),
            pl.BlockSpec((HF, HF), lambda i, j: (1, j)),
        ],
        out_specs=pl.BlockSpec((RB, HF), lambda i, j: (i + j * NRB, 0)),
        out_shape=jax.ShapeDtypeStruct((2 * NP, HF), jnp.float32),
    )(agg, agg, b1.reshape(1, H), W2, W2)


def _head_body(alo_ref, ahi_ref, b_ref, wt_ref, wb_ref, bc_ref, lp_ref, pr_ref):
    b = b_ref[...]
    zlo = jnp.maximum(alo_ref[...] + b[:, :HF], 0.0)
    zhi = jnp.maximum(ahi_ref[...] + b[:, HF:], 0.0)
    logits = (jnp.dot(zlo, wt_ref[...], preferred_element_type=jnp.float32)
              + jnp.dot(zhi, wb_ref[...], preferred_element_type=jnp.float32)
              + bc_ref[...])
    m = jnp.max(logits, axis=1, keepdims=True)
    ex = jnp.exp(logits - m)
    se = jnp.sum(ex, axis=1, keepdims=True)
    lp_ref[...] = logits - m - jnp.log(se)
    pr_ref[...] = ex / se


def _head(agg, b2, WcT, bc):
    return pl.pallas_call(
        _head_body,
        grid=(NRB,),
        in_specs=[
            pl.BlockSpec((RB, HF), lambda i: (i, 0)),
            pl.BlockSpec((RB, HF), lambda i: (i + NRB, 0)),
            pl.BlockSpec((1, H), lambda i: (0, 0)),
            pl.BlockSpec((HF, C), lambda i: (0, 0)),
            pl.BlockSpec((HF, C), lambda i: (1, 0)),
            pl.BlockSpec((1, C), lambda i: (0, 0)),
        ],
        out_specs=[
            pl.BlockSpec((RB, C), lambda i: (i, 0)),
            pl.BlockSpec((RB, C), lambda i: (i, 0)),
        ],
        out_shape=[
            jax.ShapeDtypeStruct((NP, C), jnp.float32),
            jax.ShapeDtypeStruct((NP, C), jnp.float32),
        ],
    )(agg, agg, b2.reshape(1, H), WcT, WcT, bc.reshape(1, C))


def kernel(x, edge_index, edge_weight, W1, b1, W2, b2, Wc, bc):
    src = edge_index[0]
    dst = edge_index[1]
    pad = E_PAD - E
    fill = jnp.arange(pad, dtype=jnp.int32) % N
    src_p = jnp.concatenate([src, fill]).reshape(ECH, K)
    src_p = jnp.stack([src_p, src_p + NP])
    dst_p = jnp.concatenate([dst, fill]).reshape(ECH, K)
    w_p = jnp.concatenate(
        [edge_weight, jnp.zeros((pad,), jnp.float32)]).reshape(ECH, K)
    x_pd = jnp.pad(x, ((0, NP - N), (0, 0)))

    h1 = _in_matmul(x_pd, W1)
    agg1 = _propagate(h1, src_p, dst_p, w_p)
    h2 = _mid_matmul(agg1, b1, W2)
    agg2 = _propagate(h2, src_p, dst_p, w_p)
    lp, pr = _head(agg2, b2, Wc.T, bc)
    return lp[:N], pr[:N]


# speedup vs baseline: 7.5255x; 1.0097x over previous
import functools

import jax
import jax.numpy as jnp
from jax import lax
from jax.experimental import pallas as pl
from jax.experimental.pallas import tpu as pltpu
from jax.experimental.pallas import tpu_sc as plsc

N = 10000
E = 320000
DIN = 128
H = 256
C = 64

NP = 10240
HF = H // 2
K = 96
NSC = 2
NTILE = 16
EPT = 20736
CH = EPT // K
BCH = 24
NBLK = CH // BCH
E_PAD = NTILE * EPT
ECH = E_PAD // K
SLAB = NP // NTILE
RB = 512
NRB = NP // RB


_mesh = plsc.VectorSubcoreMesh(core_axis_name="c", subcore_axis_name="s")


@functools.partial(
    pl.kernel,
    mesh=_mesh,
    out_type=jax.ShapeDtypeStruct((2 * NP, HF), jnp.float32),
    scratch_types=[
        pltpu.VMEM((BCH, K), jnp.int32),
        pltpu.VMEM((BCH, K), jnp.int32),
        pltpu.VMEM((BCH, K), jnp.float32),
        pltpu.VMEM((K, HF), jnp.float32),
        pltpu.VMEM((K, HF), jnp.float32),
        pltpu.VMEM((K, HF), jnp.float32),
        pltpu.VMEM_SHARED((NP, HF), jnp.float32),
        pltpu.SemaphoreType.DMA,
        pltpu.SemaphoreType.DMA,
        pltpu.SemaphoreType.DMA,
        pltpu.SemaphoreType.DMA,
        pltpu.SemaphoreType.DMA,
        pltpu.SemaphoreType.DMA,
    ],
)
def _propagate(h_hbm, src_hbm, dst_hbm, w_hbm, out_hbm,
               src_v, dst_v, w_v, rows0, rows1, rows2, acc_sh,
               sg0, sg1, sg2, ss0, ss1, ss2):
    c = lax.axis_index("c")
    s = lax.axis_index("s")

    def _scale(rows, j):
        @plsc.parallel_loop(0, K, step=16, unroll=3)
        def _(e0):
            wv = w_v[j, pl.ds(e0, 16)]
            for i in range(16):
                we = wv[i]
                for f in range(0, HF, 16):
                    rows[e0 + i, pl.ds(f, 16)] = (
                        rows[e0 + i, pl.ds(f, 16)] * we)

    @pl.loop(0, K)
    def _(e):
        for f in range(0, HF, 16):
            rows0[e, pl.ds(f, 16)] = jnp.zeros((16,), jnp.float32)

    for i in range(SLAB // K):
        pltpu.sync_copy(rows0, acc_sh.at[pl.ds(s * SLAB + i * K, K)])
    _rem = SLAB - (SLAB // K) * K
    if _rem:
        pltpu.sync_copy(rows0.at[pl.ds(0, _rem)],
                        acc_sh.at[pl.ds(s * SLAB + (SLAB // K) * K, _rem)])

    off = c * NP

    plsc.subcore_barrier()

    @pl.loop(0, NBLK)
    def _(t):
        base = s * CH + t * BCH
        pltpu.sync_copy(src_hbm.at[c, pl.ds(base, BCH)], src_v)
        pltpu.sync_copy(dst_hbm.at[pl.ds(base, BCH)], dst_v)
        pltpu.sync_copy(w_hbm.at[pl.ds(base, BCH)], w_v)

        bufs = ((rows0, sg0, ss0), (rows1, sg1, ss1), (rows2, sg2, ss2))
        pltpu.async_copy(h_hbm.at[src_v.at[0]], rows0, sg0)
        pltpu.async_copy(h_hbm.at[src_v.at[1]], rows1, sg1)

        @pl.loop(0, BCH, step=3)
        def _(j):
            for b in range(3):
                m = j + b
                rb, sgb, ssb = bufs[b]
                rn, sgn, ssn = bufs[(b + 2) % 3]
                pltpu.make_async_copy(h_hbm.at[src_v.at[m]], rb, sgb).wait()
                _scale(rb, m)
                pltpu.async_copy(rb, acc_sh.at[dst_v.at[m]], ssb, add=True)

                @pl.when(m + 2 < BCH)
                def _(m=m, rn=rn, sgn=sgn, ssn=ssn):
                    @pl.when(m > 0)
                    def _():
                        pltpu.make_async_copy(
                            rn, acc_sh.at[dst_v.at[m]], ssn).wait()
                    pltpu.async_copy(h_hbm.at[src_v.at[m + 2]], rn, sgn)

        pltpu.make_async_copy(rows0, acc_sh.at[dst_v.at[BCH - 3]], ss0).wait()
        pltpu.make_async_copy(rows1, acc_sh.at[dst_v.at[BCH - 2]], ss1).wait()
        pltpu.make_async_copy(rows2, acc_sh.at[dst_v.at[BCH - 1]], ss2).wait()

    plsc.subcore_barrier()

    pltpu.sync_copy(acc_sh.at[pl.ds(s * SLAB, SLAB)],
                    out_hbm.at[pl.ds(off + s * SLAB, SLAB)])


def _mm_body(x_ref, w_ref, o_ref):
    o_ref[...] = jnp.dot(x_ref[...], w_ref[...],
                         preferred_element_type=jnp.float32)


def _in_matmul(x_pd, W1):
    return pl.pallas_call(
        _mm_body,
        grid=(NRB, 2),
        in_specs=[
            pl.BlockSpec((RB, DIN), lambda i, j: (i, 0)),
            pl.BlockSpec((DIN, HF), lambda i, j: (0, j)),
        ],
        out_specs=pl.BlockSpec((RB, HF), lambda i, j: (i + j * NRB, 0)),
        out_shape=jax.ShapeDtypeStruct((2 * NP, HF), jnp.float32),
    )(x_pd, W1)


def _mid_body(alo_ref, ahi_ref, b_ref, wt_ref, wb_ref, o_ref):
    b = b_ref[...]
    zlo = jnp.maximum(alo_ref[...] + b[:, :HF], 0.0)
    zhi = jnp.maximum(ahi_ref[...] + b[:, HF:], 0.0)
    o_ref[...] = (jnp.dot(zlo, wt_ref[...], preferred_element_type=jnp.float32)
                  + jnp.dot(zhi, wb_ref[...], preferred_element_type=jnp.float32))


def _mid_matmul(agg, b1, W2):
    return pl.pallas_call(
        _mid_body,
        grid=(NRB, 2),
        in_specs=[
            pl.BlockSpec((RB, HF), lambda i, j: (i, 0)),
            pl.BlockSpec((RB, HF), lambda i, j: (i + NRB, 0)),
            pl.BlockSpec((1, H), lambda i, j: (0, 0)),
            pl.BlockSpec((HF, HF), lambda i, j: (0, j)),
            pl.BlockSpec((HF, HF), lambda i, j: (1, j)),
        ],
        out_specs=pl.BlockSpec((RB, HF), lambda i, j: (i + j * NRB, 0)),
        out_shape=jax.ShapeDtypeStruct((2 * NP, HF), jnp.float32),
    )(agg, agg, b1.reshape(1, H), W2, W2)


def _head_body(alo_ref, ahi_ref, b_ref, wt_ref, wb_ref, bc_ref, lp_ref, pr_ref):
    b = b_ref[...]
    zlo = jnp.maximum(alo_ref[...] + b[:, :HF], 0.0)
    zhi = jnp.maximum(ahi_ref[...] + b[:, HF:], 0.0)
    logits = (jnp.dot(zlo, wt_ref[...], preferred_element_type=jnp.float32)
              + jnp.dot(zhi, wb_ref[...], preferred_element_type=jnp.float32)
              + bc_ref[...])
    m = jnp.max(logits, axis=1, keepdims=True)
    ex = jnp.exp(logits - m)
    se = jnp.sum(ex, axis=1, keepdims=True)
    lp_ref[...] = logits - m - jnp.log(se)
    pr_ref[...] = ex / se


def _head(agg, b2, WcT, bc):
    return pl.pallas_call(
        _head_body,
        grid=(NRB,),
        in_specs=[
            pl.BlockSpec((RB, HF), lambda i: (i, 0)),
            pl.BlockSpec((RB, HF), lambda i: (i + NRB, 0)),
            pl.BlockSpec((1, H), lambda i: (0, 0)),
            pl.BlockSpec((HF, C), lambda i: (0, 0)),
            pl.BlockSpec((HF, C), lambda i: (1, 0)),
            pl.BlockSpec((1, C), lambda i: (0, 0)),
        ],
        out_specs=[
            pl.BlockSpec((RB, C), lambda i: (i, 0)),
            pl.BlockSpec((RB, C), lambda i: (i, 0)),
        ],
        out_shape=[
            jax.ShapeDtypeStruct((N, C), jnp.float32),
            jax.ShapeDtypeStruct((N, C), jnp.float32),
        ],
    )(agg, agg, b2.reshape(1, H), WcT, WcT, bc.reshape(1, C))


def kernel(x, edge_index, edge_weight, W1, b1, W2, b2, Wc, bc):
    src = edge_index[0]
    dst = edge_index[1]
    pad = E_PAD - E
    fill = jnp.arange(pad, dtype=jnp.int32) % N
    src_p = jnp.concatenate([src, fill]).reshape(ECH, K)
    src_p = jnp.stack([src_p, src_p + NP])
    dst_p = jnp.concatenate([dst, fill]).reshape(ECH, K)
    w_p = jnp.concatenate(
        [edge_weight, jnp.zeros((pad,), jnp.float32)]).reshape(ECH, K)

    h1 = _in_matmul(x, W1)
    agg1 = _propagate(h1, src_p, dst_p, w_p)
    h2 = _mid_matmul(agg1, b1, W2)
    agg2 = _propagate(h2, src_p, dst_p, w_p)
    lp, pr = _head(agg2, b2, Wc.T, bc)
    return lp, pr
